# baseline (speedup 1.0000x reference)
import jax
import jax.numpy as jnp
from jax import lax
from jax.experimental import pallas as pl
from jax.experimental.pallas import tpu as pltpu
from jax.experimental.pallas import tpu_sc as plsc

S = 131072
A = 4
E = 128
L = 16
NC = 2
NS = 16
NW = NC * NS
ROWS_PER_W = S // NW
C = 128
NCHUNK = ROWS_PER_W // C
EJ = E // L


def _dna_body(inputs_hbm, emb_hbm, pos_hbm, out_hbm,
              emb_v, win_v, pos_v, out_v, sem_in, sem_win, sem_out):
    wid = lax.axis_index("s") * NC + lax.axis_index("c")
    base = wid * ROWS_PER_W

    pltpu.sync_copy(emb_hbm, emb_v)
    emb_regs = [[emb_v[a, pl.ds(L * j, L)] for j in range(EJ)]
                for a in range(A)]

    WPAIR = 2 * C * A // 128

    def start_pos(g, slot):
        row0 = base + g * C
        pltpu.make_async_copy(pos_hbm.at[pl.ds(row0, C)],
                              pos_v.at[slot], sem_in.at[slot]).start()

    def start_win(p, ws):
        row0 = wid * (ROWS_PER_W * A // 128) + p * WPAIR
        pltpu.make_async_copy(inputs_hbm.at[pl.ds(row0, WPAIR)],
                              win_v.at[ws], sem_win.at[ws]).start()

    start_pos(0, 0)
    start_pos(1, 1)
    start_win(0, 0)
    start_win(1, 1)

    def chunk_body(g, _):
        slot = lax.rem(g, 2)
        parity = slot
        ws = lax.rem(g // 2, 2)

        @pl.when(g >= 2)
        def _():
            pltpu.make_async_copy(out_v.at[slot],
                                  out_hbm.at[pl.ds(base, C)],
                                  sem_out.at[slot]).wait()

        pltpu.make_async_copy(pos_hbm.at[pl.ds(base, C)],
                              pos_v.at[slot], sem_in.at[slot]).wait()

        @pl.when(parity == 0)
        def _():
            pltpu.make_async_copy(inputs_hbm.at[pl.ds(base, WPAIR)],
                                  win_v.at[ws], sem_win.at[ws]).wait()

        for jh in range(2):
            @plsc.parallel_loop(0, C // 4, unroll=2)
            def grp_body(q):
                wq = parity * 4 + q // 8
                wv = win_v[ws, wq, pl.ds((q % 8) * 16, 16)]
                for k in range(4):
                    r = q * 4 + k
                    w0 = wv[4 * k]
                    w1 = wv[4 * k + 1]
                    w2 = wv[4 * k + 2]
                    w3 = wv[4 * k + 3]
                    for j in range(jh * EJ // 2, (jh + 1) * EJ // 2):
                        t01 = w0 * emb_regs[0][j] + w1 * emb_regs[1][j]
                        t23 = w2 * emb_regs[2][j] + w3 * emb_regs[3][j]
                        out_v[slot, r, pl.ds(L * j, L)] = (
                            pos_v[slot, r, pl.ds(L * j, L)] + t01) + t23

        row0 = base + g * C
        pltpu.make_async_copy(out_v.at[slot],
                              out_hbm.at[pl.ds(row0, C)],
                              sem_out.at[slot]).start()

        @pl.when(g + 2 < NCHUNK)
        def _():
            start_pos(g + 2, slot)

        @pl.when(jnp.logical_and(parity == 1, g + 3 < NCHUNK))
        def _():
            start_win(g // 2 + 2, ws)

        return 0

    lax.fori_loop(0, NCHUNK, chunk_body, 0)

    for slot in (0, 1):
        pltpu.make_async_copy(out_v.at[slot],
                              out_hbm.at[pl.ds(base, C)],
                              sem_out.at[slot]).wait()


@jax.jit
def _dna_decode(inputs, emb_table, pos_table):
    mesh = plsc.VectorSubcoreMesh(core_axis_name="c", subcore_axis_name="s",
                                  num_cores=NC, num_subcores=NS)
    return pl.kernel(
        _dna_body,
        out_type=jax.ShapeDtypeStruct((S, E), jnp.float32),
        mesh=mesh,
        scratch_types=[
            pltpu.VMEM((A, E), jnp.float32),
            pltpu.VMEM((2, 2 * C * A // 128, 128), jnp.float32),
            pltpu.VMEM((2, C, E), jnp.float32),
            pltpu.VMEM((2, C, E), jnp.float32),
            pltpu.SemaphoreType.DMA((2,)),
            pltpu.SemaphoreType.DMA((2,)),
            pltpu.SemaphoreType.DMA((2,)),
        ],
    )(inputs, emb_table, pos_table)


def kernel(inputs, emb_table, pos_table):
    inputs = inputs.reshape(S * A // 128, 128)
    return _dna_decode(inputs, emb_table, pos_table)


# speedup vs baseline: 2.3675x; 2.3675x over previous
import jax
import jax.numpy as jnp
from jax import lax
from jax.experimental import pallas as pl
from jax.experimental.pallas import tpu as pltpu
from jax.experimental.pallas import tpu_sc as plsc

S = 131072
A = 4
E = 128
L = 16
NC = 2
NS = 16
NW = NC * NS
ROWS_PER_W = S // NW
C = 128
NCHUNK = ROWS_PER_W // C
EJ = E // L
NSLOT = 4


def _dna_body(inputs_hbm, emb_hbm, pos_hbm, out_hbm,
              emb_v, win_v, out_v, sem_in, sem_win, sem_out):
    wid = lax.axis_index("s") * NC + lax.axis_index("c")
    base = wid * ROWS_PER_W

    pltpu.sync_copy(emb_hbm, emb_v)

    WPAIR = 2 * C * A // 128

    def start_pos(g, slot):
        row0 = base + g * C
        pltpu.make_async_copy(pos_hbm.at[pl.ds(row0, C)],
                              out_v.at[slot], sem_in.at[slot]).start()

    def start_win(p, ws):
        row0 = wid * (ROWS_PER_W * A // 128) + p * WPAIR
        pltpu.make_async_copy(inputs_hbm.at[pl.ds(row0, WPAIR)],
                              win_v.at[ws], sem_win.at[ws]).start()

    start_pos(0, 0)
    start_pos(1, 1)
    start_win(0, 0)
    start_win(1, 1)

    def chunk_body(g, _):
        slot = lax.rem(g, NSLOT)
        parity = lax.rem(g, 2)
        ws = lax.rem(g // 2, 2)

        @pl.when(g + 2 < NCHUNK)
        def _():
            nslot = lax.rem(g + 2, NSLOT)

            @pl.when(g >= 2)
            def _():
                pltpu.make_async_copy(out_v.at[nslot],
                                      out_hbm.at[pl.ds(base, C)],
                                      sem_out.at[nslot]).wait()

            start_pos(g + 2, nslot)

        pltpu.make_async_copy(pos_hbm.at[pl.ds(base, C)],
                              out_v.at[slot], sem_in.at[slot]).wait()

        @pl.when(parity == 0)
        def _():
            pltpu.make_async_copy(inputs_hbm.at[pl.ds(base, WPAIR)],
                                  win_v.at[ws], sem_win.at[ws]).wait()

        wbase = parity * 4

        @plsc.parallel_loop(0, C // 16, unroll=1)
        def grp_body(m):
            wvs = [win_v[ws, wbase + a, pl.ds(16 * m, 16)]
                   for a in range(A)]
            for jh in range(2):
                jlo = jh * EJ // 2
                emb_regs = [[emb_v[a, pl.ds(L * (jlo + jj), L)]
                             for jj in range(EJ // 2)] for a in range(A)]
                for k in range(16):
                    r = m * 16 + k
                    w0 = wvs[0][k]
                    w1 = wvs[1][k]
                    w2 = wvs[2][k]
                    w3 = wvs[3][k]
                    for jj in range(EJ // 2):
                        j = jlo + jj
                        t01 = w0 * emb_regs[0][jj] + w1 * emb_regs[1][jj]
                        t23 = w2 * emb_regs[2][jj] + w3 * emb_regs[3][jj]
                        plsc.addupdate(
                            out_v.at[slot, r, pl.ds(L * j, L)], t01 + t23)

        row0 = base + g * C
        pltpu.make_async_copy(out_v.at[slot],
                              out_hbm.at[pl.ds(row0, C)],
                              sem_out.at[slot]).start()

        @pl.when(jnp.logical_and(parity == 1, g + 3 < NCHUNK))
        def _():
            start_win(g // 2 + 2, ws)

        return 0

    lax.fori_loop(0, NCHUNK, chunk_body, 0)

    for slot in range(NSLOT - 2):
        pltpu.make_async_copy(out_v.at[slot],
                              out_hbm.at[pl.ds(base, C)],
                              sem_out.at[slot]).wait()
    for slot in range(NSLOT - 2, NSLOT):
        pltpu.make_async_copy(out_v.at[slot],
                              out_hbm.at[pl.ds(base, C)],
                              sem_out.at[slot]).wait()


@jax.jit
def _dna_decode(inputs, emb_table, pos_table):
    mesh = plsc.VectorSubcoreMesh(core_axis_name="c", subcore_axis_name="s",
                                  num_cores=NC, num_subcores=NS)
    return pl.kernel(
        _dna_body,
        out_type=jax.ShapeDtypeStruct((S, E), jnp.float32),
        mesh=mesh,
        scratch_types=[
            pltpu.VMEM((A, E), jnp.float32),
            pltpu.VMEM((2, 2 * C * A // 128, 128), jnp.float32),
            pltpu.VMEM((NSLOT, C, E), jnp.float32),
            pltpu.SemaphoreType.DMA((NSLOT,)),
            pltpu.SemaphoreType.DMA((2,)),
            pltpu.SemaphoreType.DMA((NSLOT,)),
        ],
    )(inputs, emb_table, pos_table)


def kernel(inputs, emb_table, pos_table):
    w_planes = (inputs.reshape(S // 128, 128, A)
                .transpose(0, 2, 1)
                .reshape(S * A // 128, 128))
    return _dna_decode(w_planes, emb_table, pos_table)
